# Initial kernel scaffold; baseline (speedup 1.0000x reference)
import functools

import jax
import jax.numpy as jnp
from jax import lax
from jax.experimental import pallas as pl
from jax.experimental.pallas import tpu as pltpu
from jax.experimental.pallas import tpu_sc as plsc

_NC = 2
_NS = 16
_NW = _NC * _NS


@functools.lru_cache(maxsize=None)
def _gather2_kernel(b_flat: int, d: int):
    b_per_w = b_flat // _NW
    assert b_flat % (8 * _NW) == 0
    mesh = plsc.VectorSubcoreMesh(core_axis_name="c", subcore_axis_name="s")

    @functools.partial(
        pl.kernel,
        mesh=mesh,
        out_type=(
            jax.ShapeDtypeStruct((b_flat, d), jnp.float32),
            jax.ShapeDtypeStruct((b_flat, d), jnp.float32),
        ),
        scratch_types=[
            pltpu.VMEM((b_per_w,), jnp.int32),
            pltpu.VMEM((b_per_w, d), jnp.float32),
            pltpu.SemaphoreType.DMA,
        ],
    )
    def k(dx_idx_hbm, proc_idx_hbm, dx_tab_hbm, proc_tab_hbm,
          dx_out_hbm, proc_out_hbm, idx_v, rows_v, sem):
        wid = lax.axis_index("s") * _NC + lax.axis_index("c")
        base = wid * b_per_w
        pltpu.sync_copy(dx_idx_hbm.at[pl.ds(base, b_per_w)], idx_v)
        pltpu.async_copy(dx_tab_hbm.at[idx_v], rows_v, sem).wait()
        pltpu.sync_copy(rows_v, dx_out_hbm.at[pl.ds(base, b_per_w)])
        pltpu.sync_copy(proc_idx_hbm.at[pl.ds(base, b_per_w)], idx_v)
        pltpu.async_copy(proc_tab_hbm.at[idx_v], rows_v, sem).wait()
        pltpu.sync_copy(rows_v, proc_out_hbm.at[pl.ds(base, b_per_w)])

    return k


def kernel(dx_ints, proc_ints, dx_table, proc_table, visit_param, max_num_codes):
    b, l = dx_ints.shape
    d = dx_table.shape[1]
    b_flat = b * l
    gather2 = _gather2_kernel(b_flat, d)
    emb_dx, emb_proc = gather2(
        dx_ints.reshape(b_flat), proc_ints.reshape(b_flat), dx_table, proc_table
    )
    emb_dx = emb_dx.reshape(b, l, d)
    emb_proc = emb_proc.reshape(b, l, d)
    mask_dx = jnp.ones((b, l, 1), dtype=jnp.float32)
    mask_proc = jnp.ones((b, l, 1), dtype=jnp.float32)
    visit_emb = jnp.broadcast_to(visit_param[None, :, :], (1, 1, d))
    mask_visit = jnp.ones((1, 1), dtype=jnp.float32)
    return (emb_dx, emb_proc, visit_emb, mask_dx, mask_proc, mask_visit)


# speedup vs baseline: 3.9839x; 3.9839x over previous
import functools

import jax
import jax.numpy as jnp
from jax import lax
from jax.experimental import pallas as pl
from jax.experimental.pallas import tpu as pltpu
from jax.experimental.pallas import tpu_sc as plsc

_NC = 2
_NS = 16
_NW = _NC * _NS


@functools.lru_cache(maxsize=None)
def _gather2_kernel(b_flat: int, d: int):
    b_per_w = b_flat // _NW
    assert b_flat % (8 * _NW) == 0
    mesh = plsc.VectorSubcoreMesh(core_axis_name="c", subcore_axis_name="s")

    @functools.partial(
        pl.kernel,
        mesh=mesh,
        out_type=(
            jax.ShapeDtypeStruct((b_flat, d), jnp.float32),
            jax.ShapeDtypeStruct((b_flat, d), jnp.float32),
        ),
        scratch_types=[
            pltpu.VMEM((b_per_w,), jnp.int32),
            pltpu.VMEM((b_per_w, d), jnp.float32),
            pltpu.SemaphoreType.DMA,
        ],
        compiler_params=pltpu.CompilerParams(use_tc_tiling_on_sc=False),
    )
    def k(dx_idx_hbm, proc_idx_hbm, dx_tab_hbm, proc_tab_hbm,
          dx_out_hbm, proc_out_hbm, idx_v, rows_v, sem):
        wid = lax.axis_index("s") * _NC + lax.axis_index("c")
        base = wid * b_per_w
        pltpu.sync_copy(dx_idx_hbm.at[pl.ds(base, b_per_w)], idx_v)
        pltpu.async_copy(dx_tab_hbm.at[idx_v], rows_v, sem).wait()
        pltpu.sync_copy(rows_v, dx_out_hbm.at[pl.ds(base, b_per_w)])
        pltpu.sync_copy(proc_idx_hbm.at[pl.ds(base, b_per_w)], idx_v)
        pltpu.async_copy(proc_tab_hbm.at[idx_v], rows_v, sem).wait()
        pltpu.sync_copy(rows_v, proc_out_hbm.at[pl.ds(base, b_per_w)])

    return k


def kernel(dx_ints, proc_ints, dx_table, proc_table, visit_param, max_num_codes):
    b, l = dx_ints.shape
    d = dx_table.shape[1]
    b_flat = b * l
    gather2 = _gather2_kernel(b_flat, d)
    emb_dx, emb_proc = gather2(
        dx_ints.reshape(b_flat), proc_ints.reshape(b_flat), dx_table, proc_table
    )
    emb_dx = emb_dx.reshape(b, l, d)
    emb_proc = emb_proc.reshape(b, l, d)
    mask_dx = jnp.ones((b, l, 1), dtype=jnp.float32)
    mask_proc = jnp.ones((b, l, 1), dtype=jnp.float32)
    visit_emb = jnp.broadcast_to(visit_param[None, :, :], (1, 1, d))
    mask_visit = jnp.ones((1, 1), dtype=jnp.float32)
    return (emb_dx, emb_proc, visit_emb, mask_dx, mask_proc, mask_visit)


# speedup vs baseline: 4.4466x; 1.1161x over previous
import functools

import jax
import jax.numpy as jnp
from jax import lax
from jax.experimental import pallas as pl
from jax.experimental.pallas import tpu as pltpu
from jax.experimental.pallas import tpu_sc as plsc

_NC = 2
_NS = 16
_NW = _NC * _NS
_CHUNK = 1024


@functools.lru_cache(maxsize=None)
def _gather2_kernel(b_flat: int, v: int, d: int):
    n_chunks = b_flat // _CHUNK
    max_per_w = -(-n_chunks // _NW)
    out_rows = b_flat * d // 128
    rpc = _CHUNK * d // 128
    mesh = plsc.VectorSubcoreMesh(core_axis_name="c", subcore_axis_name="s")

    @functools.partial(
        pl.kernel,
        mesh=mesh,
        out_type=(
            jax.ShapeDtypeStruct((out_rows, 128), jnp.float32),
            jax.ShapeDtypeStruct((out_rows, 128), jnp.float32),
        ),
        scratch_types=[
            pltpu.VMEM((8, 128), jnp.int32),
            pltpu.VMEM((8, 128, d), jnp.float32),
            pltpu.SemaphoreType.DMA,
            pltpu.SemaphoreType.DMA,
        ],
        compiler_params=pltpu.CompilerParams(use_tc_tiling_on_sc=False),
    )
    def k(dx_idx_hbm, proc_idx_hbm, dx_tab_hbm, proc_tab_hbm,
          dx_out_hbm, proc_out_hbm, idx_v, rows_v, gsem, osem):
        wid = lax.axis_index("s") * _NC + lax.axis_index("c")

        def do_table(idx_hbm, tab_hbm, out_hbm, c):
            pltpu.sync_copy(idx_hbm.at[c], idx_v)
            for s in range(8):
                pltpu.make_async_copy(
                    tab_hbm.at[idx_v.at[s]], rows_v.at[s], gsem
                ).start()
            for s in range(8):
                pltpu.make_async_copy(
                    tab_hbm.at[idx_v.at[s]], rows_v.at[s], gsem
                ).wait()
            for s in range(8):
                pltpu.make_async_copy(
                    rows_v.at[s],
                    out_hbm.at[pl.ds(c * rpc, rpc), pl.ds(d * s, d)],
                    osem,
                ).start()
            for s in range(8):
                pltpu.make_async_copy(
                    rows_v.at[s],
                    out_hbm.at[pl.ds(c * rpc, rpc), pl.ds(d * s, d)],
                    osem,
                ).wait()

        def body(i, _):
            c = wid + i * _NW

            @pl.when(c < n_chunks)
            def _():
                do_table(dx_idx_hbm, dx_tab_hbm, dx_out_hbm, c)
                do_table(proc_idx_hbm, proc_tab_hbm, proc_out_hbm, c)
            return 0

        lax.fori_loop(0, max_per_w, body, 0)

    return k


def kernel(dx_ints, proc_ints, dx_table, proc_table, visit_param, max_num_codes):
    b, l = dx_ints.shape
    v = dx_table.shape[0]
    d = dx_table.shape[1]
    b_flat = b * l
    n_chunks = b_flat // _CHUNK
    gather2 = _gather2_kernel(b_flat, v, d)
    dx_idx3 = dx_ints.reshape(n_chunks, 128, 8).transpose(0, 2, 1)
    proc_idx3 = proc_ints.reshape(n_chunks, 128, 8).transpose(0, 2, 1)
    emb_dx128, emb_proc128 = gather2(dx_idx3, proc_idx3, dx_table, proc_table)
    emb_dx = emb_dx128.reshape(b, l, d)
    emb_proc = emb_proc128.reshape(b, l, d)
    mask_dx = jnp.ones((b, l, 1), dtype=jnp.float32)
    mask_proc = jnp.ones((b, l, 1), dtype=jnp.float32)
    visit_emb = jnp.broadcast_to(visit_param[None, :, :], (1, 1, d))
    mask_visit = jnp.ones((1, 1), dtype=jnp.float32)
    return (emb_dx, emb_proc, visit_emb, mask_dx, mask_proc, mask_visit)


# speedup vs baseline: 4.6797x; 1.0524x over previous
import functools

import jax
import jax.numpy as jnp
from jax import lax
from jax.experimental import pallas as pl
from jax.experimental.pallas import tpu as pltpu
from jax.experimental.pallas import tpu_sc as plsc

_NC = 2
_NS = 16
_NW = _NC * _NS
_CHUNK = 1024


@functools.lru_cache(maxsize=None)
def _gather2_kernel(b_flat: int, v: int, d: int):
    n_chunks = b_flat // _CHUNK
    max_per_w = -(-n_chunks // _NW)
    out_rows = b_flat * d // 128
    rpc = _CHUNK * d // 128
    mesh = plsc.VectorSubcoreMesh(core_axis_name="c", subcore_axis_name="s")

    @functools.partial(
        pl.kernel,
        mesh=mesh,
        out_type=(
            jax.ShapeDtypeStruct((out_rows, 128), jnp.float32),
            jax.ShapeDtypeStruct((out_rows, 128), jnp.float32),
        ),
        scratch_types=[
            pltpu.VMEM((_CHUNK,), jnp.int32),
            pltpu.VMEM((8, 128), jnp.int32),
            pltpu.VMEM((8, 128, d), jnp.float32),
            pltpu.SemaphoreType.DMA,
            pltpu.SemaphoreType.DMA,
        ],
        compiler_params=pltpu.CompilerParams(
            use_tc_tiling_on_sc=False, needs_layout_passes=False
        ),
    )
    def k(dx_idx_hbm, proc_idx_hbm, dx_tab_hbm, proc_tab_hbm,
          dx_out_hbm, proc_out_hbm, raw_v, idx_v, rows_v, gsem, osem):
        wid = lax.axis_index("s") * _NC + lax.axis_index("c")
        lane = lax.iota(jnp.int32, 16)

        def do_table(idx_hbm, tab_hbm, out_hbm, c):
            pltpu.sync_copy(idx_hbm.at[pl.ds(c * _CHUNK, _CHUNK)], raw_v)
            for s in range(8):
                for kk in range(8):
                    offs = lane * 8 + (128 * kk + s)
                    vals = plsc.load_gather(raw_v, [offs])
                    idx_v[s, pl.ds(16 * kk, 16)] = vals
            for s in range(8):
                pltpu.make_async_copy(
                    tab_hbm.at[idx_v.at[s]], rows_v.at[s], gsem
                ).start()
            for s in range(8):
                pltpu.make_async_copy(
                    tab_hbm.at[idx_v.at[s]], rows_v.at[s], gsem
                ).wait()
            for s in range(8):
                pltpu.make_async_copy(
                    rows_v.at[s],
                    out_hbm.at[pl.ds(c * rpc, rpc), pl.ds(d * s, d)],
                    osem,
                ).start()
            for s in range(8):
                pltpu.make_async_copy(
                    rows_v.at[s],
                    out_hbm.at[pl.ds(c * rpc, rpc), pl.ds(d * s, d)],
                    osem,
                ).wait()

        def body(i, _):
            c = wid + i * _NW

            @pl.when(c < n_chunks)
            def _():
                do_table(dx_idx_hbm, dx_tab_hbm, dx_out_hbm, c)
                do_table(proc_idx_hbm, proc_tab_hbm, proc_out_hbm, c)
            return 0

        lax.fori_loop(0, max_per_w, body, 0)

    return k


def kernel(dx_ints, proc_ints, dx_table, proc_table, visit_param, max_num_codes):
    b, l = dx_ints.shape
    v = dx_table.shape[0]
    d = dx_table.shape[1]
    b_flat = b * l
    gather2 = _gather2_kernel(b_flat, v, d)
    emb_dx128, emb_proc128 = gather2(
        dx_ints.reshape(b_flat), proc_ints.reshape(b_flat), dx_table, proc_table
    )
    emb_dx = emb_dx128.reshape(b, l, d)
    emb_proc = emb_proc128.reshape(b, l, d)
    mask_dx = jnp.ones((b, l, 1), dtype=jnp.float32)
    mask_proc = jnp.ones((b, l, 1), dtype=jnp.float32)
    visit_emb = jnp.broadcast_to(visit_param[None, :, :], (1, 1, d))
    mask_visit = jnp.ones((1, 1), dtype=jnp.float32)
    return (emb_dx, emb_proc, visit_emb, mask_dx, mask_proc, mask_visit)


# speedup vs baseline: 4.9762x; 1.0634x over previous
import functools

import jax
import jax.numpy as jnp
from jax import lax
from jax.experimental import pallas as pl
from jax.experimental.pallas import tpu as pltpu
from jax.experimental.pallas import tpu_sc as plsc

_NC = 2
_NS = 16
_NW = _NC * _NS


@functools.lru_cache(maxsize=None)
def _gather2_kernel(b_flat: int, v: int, d: int):
    b_per_w = b_flat // _NW
    sub = b_per_w // 8
    out_rows = b_flat * d // 128
    orow_per_w = b_per_w * d // 128
    mesh = plsc.VectorSubcoreMesh(core_axis_name="c", subcore_axis_name="s")

    @functools.partial(
        pl.kernel,
        mesh=mesh,
        out_type=(
            jax.ShapeDtypeStruct((out_rows, 128), jnp.float32),
            jax.ShapeDtypeStruct((out_rows, 128), jnp.float32),
        ),
        scratch_types=[
            pltpu.VMEM((b_per_w,), jnp.int32),
            pltpu.VMEM((8, sub), jnp.int32),
            pltpu.VMEM((8, sub, d), jnp.float32),
            pltpu.SemaphoreType.DMA,
            pltpu.SemaphoreType.DMA,
        ],
        compiler_params=pltpu.CompilerParams(
            use_tc_tiling_on_sc=False, needs_layout_passes=False
        ),
    )
    def k(dx_idx_hbm, proc_idx_hbm, dx_tab_hbm, proc_tab_hbm,
          dx_out_hbm, proc_out_hbm, raw_v, idx_v, rows_v, gsem, osem):
        wid = lax.axis_index("s") * _NC + lax.axis_index("c")
        base = wid * b_per_w
        obase = wid * orow_per_w
        lane = lax.iota(jnp.int32, 16)

        def do_table(idx_hbm, tab_hbm, out_hbm):
            pltpu.sync_copy(idx_hbm.at[pl.ds(base, b_per_w)], raw_v)

            def reorder(g, _):
                offs = g * 128 + lane * 8
                for s in range(8):
                    idx_v[s, pl.ds(g * 16, 16)] = plsc.load_gather(
                        raw_v, [offs + s]
                    )
                return 0

            lax.fori_loop(0, sub // 16, reorder, 0)

            for s in range(8):
                pltpu.make_async_copy(
                    tab_hbm.at[idx_v.at[s]], rows_v.at[s], gsem
                ).start()
            for s in range(8):
                pltpu.make_async_copy(
                    tab_hbm.at[idx_v.at[s]], rows_v.at[s], gsem
                ).wait()
            for s in range(8):
                pltpu.make_async_copy(
                    rows_v.at[s],
                    out_hbm.at[pl.ds(obase, orow_per_w), pl.ds(d * s, d)],
                    osem,
                ).start()
            for s in range(8):
                pltpu.make_async_copy(
                    rows_v.at[s],
                    out_hbm.at[pl.ds(obase, orow_per_w), pl.ds(d * s, d)],
                    osem,
                ).wait()

        do_table(dx_idx_hbm, dx_tab_hbm, dx_out_hbm)
        do_table(proc_idx_hbm, proc_tab_hbm, proc_out_hbm)

    return k


def kernel(dx_ints, proc_ints, dx_table, proc_table, visit_param, max_num_codes):
    b, l = dx_ints.shape
    v = dx_table.shape[0]
    d = dx_table.shape[1]
    b_flat = b * l
    gather2 = _gather2_kernel(b_flat, v, d)
    emb_dx128, emb_proc128 = gather2(
        dx_ints.reshape(b_flat), proc_ints.reshape(b_flat), dx_table, proc_table
    )
    emb_dx = emb_dx128.reshape(b, l, d)
    emb_proc = emb_proc128.reshape(b, l, d)
    mask_dx = jnp.ones((b, l, 1), dtype=jnp.float32)
    mask_proc = jnp.ones((b, l, 1), dtype=jnp.float32)
    visit_emb = jnp.broadcast_to(visit_param[None, :, :], (1, 1, d))
    mask_visit = jnp.ones((1, 1), dtype=jnp.float32)
    return (emb_dx, emb_proc, visit_emb, mask_dx, mask_proc, mask_visit)
